# baseline (speedup 1.0000x reference)
import functools

import jax
import jax.numpy as jnp
from jax import lax
from jax.experimental import pallas as pl
from jax.experimental.pallas import tpu as pltpu
from jax.experimental.pallas import tpu_sc as plsc

NE = 8
TK = 2
BLK = 128
S = 2048
H = 768
I = 1536
NSLOT = S * TK
PAD = NSLOT + NE * BLK
NB = PAD // BLK
LANES = 128
WLANE = 128
NTILE = 32
RING = 2


def _router_body(x_ref, gw_ref, logits_ref, p1_ref, p2_ref, w1_ref, w2_ref,
                 be_ref, vb_ref):
    x = x_ref[...]
    gw = gw_ref[...]
    logits = lax.dot_general(x, gw, (((1,), (1,)), ((), ())),
                             preferred_element_type=jnp.float32)
    logits_ref[...] = logits
    m = jnp.max(logits, axis=-1, keepdims=True)
    ex = jnp.exp(logits - m)
    p = ex / jnp.sum(ex, axis=-1, keepdims=True)
    idx = lax.broadcasted_iota(jnp.int32, p.shape, 1)
    m1 = jnp.max(p, axis=-1, keepdims=True)
    i1 = jnp.min(jnp.where(p == m1, idx, NE), axis=-1, keepdims=True)
    pm = jnp.where(idx == i1, -1.0, p)
    m2 = jnp.max(pm, axis=-1, keepdims=True)
    i2 = jnp.min(jnp.where(pm == m2, idx, NE), axis=-1, keepdims=True)
    denom = m1 + m2
    w1 = m1 / denom
    w2 = m2 / denom

    oh1 = (idx == i1).astype(jnp.float32)
    oh2 = (idx == i2).astype(jnp.float32)
    a = oh1 + oh2
    r_io = lax.broadcasted_iota(jnp.int32, (BLK, BLK), 0)
    c_io = lax.broadcasted_iota(jnp.int32, (BLK, BLK), 1)
    tri = (r_io >= c_io).astype(jnp.float32)
    chunks = []
    run = jnp.zeros((1, NE), jnp.float32)
    for c in range(S // BLK):
        a_c = lax.slice(a, (c * BLK, 0), ((c + 1) * BLK, NE))
        cum_c = lax.dot_general(tri, a_c, (((1,), (0,)), ((), ())),
                                preferred_element_type=jnp.float32) + run
        chunks.append(cum_c)
        run = lax.slice(cum_c, (BLK - 1, 0), (BLK, NE))
    cum_incl = jnp.concatenate(chunks, axis=0)
    cum_excl = cum_incl - a
    counts = run
    padded = jnp.maximum(
        jnp.floor((counts + (BLK - 1)) * (1.0 / BLK)) * BLK, float(BLK))
    r8 = lax.broadcasted_iota(jnp.int32, (NE, NE), 0)
    c8 = lax.broadcasted_iota(jnp.int32, (NE, NE), 1)
    triu = (r8 < c8).astype(jnp.float32)
    pad_off = lax.dot_general(padded, triu, (((1,), (0,)), ((), ())),
                              preferred_element_type=jnp.float32)
    pos_base = pad_off + cum_excl
    pos1 = jnp.sum(oh1 * pos_base, axis=-1, keepdims=True)
    pos2 = jnp.sum(oh2 * pos_base, axis=-1, keepdims=True)

    p1_ref[...] = jnp.reshape(pos1.astype(jnp.int32), (NTILE, S // NTILE))
    p2_ref[...] = jnp.reshape(pos2.astype(jnp.int32), (NTILE, S // NTILE))
    w1_ref[...] = jnp.broadcast_to(w1, (S, WLANE))
    w2_ref[...] = jnp.broadcast_to(w2, (S, WLANE))
    pad_end = pad_off + padded
    starts = (lax.broadcasted_iota(jnp.int32, (64, 1), 0) * BLK
              ).astype(jnp.float32)
    be = jnp.sum((jnp.broadcast_to(pad_end, (64, NE)) <= starts)
                 .astype(jnp.float32), axis=1, keepdims=True)
    be_ref[...] = jnp.minimum(be, NE - 1).astype(jnp.int32)
    nv = jnp.sum(padded, axis=1, keepdims=True) * (1.0 / BLK)
    bix = (lax.broadcasted_iota(jnp.int32, (64, 1), 0)
           ).astype(jnp.float32)
    vb_ref[...] = jnp.minimum(bix, nv - 1.0).astype(jnp.int32)


def _run_router(x, gate_w):
    return pl.pallas_call(
        _router_body,
        out_shape=(
            jax.ShapeDtypeStruct((S, NE), jnp.float32),
            jax.ShapeDtypeStruct((NTILE, S // NTILE), jnp.int32),
            jax.ShapeDtypeStruct((NTILE, S // NTILE), jnp.int32),
            jax.ShapeDtypeStruct((S, WLANE), jnp.float32),
            jax.ShapeDtypeStruct((S, WLANE), jnp.float32),
            jax.ShapeDtypeStruct((64, 1), jnp.int32),
            jax.ShapeDtypeStruct((64, 1), jnp.int32),
        ),
    )(x, gate_w)


def _gmm_body(be_ref, vb_ref, xg_ref, gw_hbm, uw_hbm, dw_hbm, wcol_ref,
              out_ref, gbuf, ubuf, dbuf, gsem, usem, dsem):
    b = pl.program_id(0)
    e = be_ref[b, 0]
    prev_e = be_ref[jnp.maximum(b - 1, 0), 0]
    flo = jnp.where(b == 0, 0, prev_e + RING)
    fhi = jnp.minimum(e + RING, NE)
    wlo = jnp.where(b == 0, 0, prev_e + 1)
    whi = e + 1
    for k in range(NE):
        @pl.when((k >= flo) & (k < fhi))
        def _(k=k):
            pltpu.make_async_copy(gw_hbm.at[k], gbuf.at[k % RING],
                                  gsem.at[k % RING]).start()
            pltpu.make_async_copy(uw_hbm.at[k], ubuf.at[k % RING],
                                  usem.at[k % RING]).start()
            pltpu.make_async_copy(dw_hbm.at[k], dbuf.at[k % RING],
                                  dsem.at[k % RING]).start()
    for k in range(NE):
        @pl.when((k >= wlo) & (k < whi))
        def _(k=k):
            pltpu.make_async_copy(gw_hbm.at[k], gbuf.at[k % RING],
                                  gsem.at[k % RING]).wait()
            pltpu.make_async_copy(uw_hbm.at[k], ubuf.at[k % RING],
                                  usem.at[k % RING]).wait()
            pltpu.make_async_copy(dw_hbm.at[k], dbuf.at[k % RING],
                                  dsem.at[k % RING]).wait()

    @pl.when(vb_ref[b, 0] == b)
    def _():
        slot = e % RING
        x = xg_ref[...]
        gw = gbuf[slot]
        uw = ubuf[slot]
        dw = dbuf[slot]
        g = lax.dot_general(x, gw, (((1,), (1,)), ((), ())),
                            preferred_element_type=jnp.float32)
        u = lax.dot_general(x, uw, (((1,), (1,)), ((), ())),
                            preferred_element_type=jnp.float32)
        h = g * jax.nn.sigmoid(g) * u
        y = lax.dot_general(h, dw, (((1,), (1,)), ((), ())),
                            preferred_element_type=jnp.float32)
        out_ref[...] = y * wcol_ref[...][:, :1]


def _run_gmm(block_expert, vblock, xg, gate_proj_w, up_proj_w, down_proj_w,
             wmat):
    grid_spec = pltpu.PrefetchScalarGridSpec(
        num_scalar_prefetch=2,
        grid=(NB,),
        in_specs=[
            pl.BlockSpec((BLK, H), lambda b, be, vb: (vb[b, 0], 0)),
            pl.BlockSpec(memory_space=pl.ANY),
            pl.BlockSpec(memory_space=pl.ANY),
            pl.BlockSpec(memory_space=pl.ANY),
            pl.BlockSpec((BLK, WLANE), lambda b, be, vb: (vb[b, 0], 0)),
        ],
        out_specs=pl.BlockSpec((BLK, H), lambda b, be, vb: (vb[b, 0], 0)),
        scratch_shapes=[
            pltpu.VMEM((RING, I, H), jnp.float32),
            pltpu.VMEM((RING, I, H), jnp.float32),
            pltpu.VMEM((RING, H, I), jnp.float32),
            pltpu.SemaphoreType.DMA((RING,)),
            pltpu.SemaphoreType.DMA((RING,)),
            pltpu.SemaphoreType.DMA((RING,)),
        ],
    )
    return pl.pallas_call(
        _gmm_body,
        grid_spec=grid_spec,
        out_shape=jax.ShapeDtypeStruct((PAD, H), jnp.float32),
    )(block_expert, vblock, xg, gate_proj_w, up_proj_w, down_proj_w, wmat)


def _sc_dispatch(x, p1r, p2r, w1r, w2r):
    mesh = plsc.VectorSubcoreMesh(core_axis_name="c", subcore_axis_name="s")
    nw = mesh.num_cores * mesh.num_subcores
    t_per_w = S // nw

    @functools.partial(
        pl.kernel,
        out_type=(
            jax.ShapeDtypeStruct((PAD, H), jnp.float32),
            jax.ShapeDtypeStruct((PAD, WLANE), jnp.float32),
        ),
        mesh=mesh,
        name="sc_dispatch",
        scratch_types=[
            pltpu.VMEM((t_per_w, H), jnp.float32),
            pltpu.VMEM((t_per_w, WLANE), jnp.float32),
            pltpu.VMEM((t_per_w, WLANE), jnp.float32),
            pltpu.VMEM((t_per_w,), jnp.int32),
            pltpu.VMEM((t_per_w,), jnp.int32),
            pltpu.SemaphoreType.DMA,
            pltpu.SemaphoreType.DMA,
            pltpu.SemaphoreType.DMA,
            pltpu.SemaphoreType.DMA,
        ],
    )
    def k(x_hbm, p1_hbm, p2_hbm, w1_hbm, w2_hbm, xg_hbm, wm_hbm,
          xrows_v, w1m_v, w2m_v, p1_v, p2_v,
          s1, s2, s3, s4):
        wid = lax.axis_index("s") * mesh.num_cores + lax.axis_index("c")
        base = wid * t_per_w
        sl = pl.ds(base, t_per_w)
        pltpu.sync_copy(x_hbm.at[sl], xrows_v)
        pltpu.sync_copy(p1_hbm.at[wid], p1_v)
        pltpu.sync_copy(p2_hbm.at[wid], p2_v)
        pltpu.sync_copy(w1_hbm.at[sl], w1m_v)
        pltpu.sync_copy(w2_hbm.at[sl], w2m_v)
        c1 = pltpu.async_copy(xrows_v, xg_hbm.at[p1_v], s1)
        c2 = pltpu.async_copy(xrows_v, xg_hbm.at[p2_v], s2)
        c3 = pltpu.async_copy(w1m_v, wm_hbm.at[p1_v], s3)
        c4 = pltpu.async_copy(w2m_v, wm_hbm.at[p2_v], s4)
        c1.wait()
        c2.wait()
        c3.wait()
        c4.wait()

    return k(x, p1r, p2r, w1r, w2r)


def _sc_combine(osort, p1r, p2r):
    mesh = plsc.VectorSubcoreMesh(core_axis_name="c", subcore_axis_name="s")
    nw = mesh.num_cores * mesh.num_subcores
    t_per_w = S // nw
    csteps = H // 16

    @functools.partial(
        pl.kernel,
        out_type=jax.ShapeDtypeStruct((S, H), jnp.float32),
        mesh=mesh,
        name="sc_combine_rows",
        scratch_types=[
            pltpu.VMEM((t_per_w,), jnp.int32),
            pltpu.VMEM((t_per_w,), jnp.int32),
            pltpu.VMEM((t_per_w, H), jnp.float32),
            pltpu.VMEM((t_per_w, H), jnp.float32),
            pltpu.SemaphoreType.DMA,
        ],
    )
    def k(osort_hbm, p1_hbm, p2_hbm, out_hbm,
          i0_v, i1_v, a_v, b_v, sem):
        wid = lax.axis_index("s") * mesh.num_cores + lax.axis_index("c")
        base = wid * t_per_w
        sl = pl.ds(base, t_per_w)
        pltpu.sync_copy(p1_hbm.at[wid], i0_v)
        pltpu.sync_copy(p2_hbm.at[wid], i1_v)
        cp0 = pltpu.async_copy(osort_hbm.at[i0_v], a_v, sem)
        cp1 = pltpu.async_copy(osort_hbm.at[i1_v], b_v, sem)
        cp0.wait()
        cp1.wait()

        @plsc.parallel_loop(0, t_per_w * csteps, unroll=8)
        def _add(i):
            r = i // csteps
            c = i - r * csteps
            cs = pl.ds(c * 16, 16)
            a_v[r, cs] = a_v[r, cs] + b_v[r, cs]

        pltpu.sync_copy(a_v, out_hbm.at[sl])

    return k(osort, p1r, p2r)


def kernel(hidden_states, gate_w, gate_proj_w, up_proj_w, down_proj_w):
    batch, seq, hdim = hidden_states.shape
    x = hidden_states.reshape(-1, hdim)

    logits, p1r, p2r, w1r, w2r, block_expert, vblock = _run_router(x, gate_w)

    xg, wmat = _sc_dispatch(x, p1r, p2r, w1r, w2r)
    osort = _run_gmm(block_expert, vblock, xg, gate_proj_w, up_proj_w,
                     down_proj_w, wmat)
    final = _sc_combine(osort, p1r, p2r)
    return final.reshape(batch, seq, hdim), logits


# speedup vs baseline: 1.0142x; 1.0142x over previous
import functools

import jax
import jax.numpy as jnp
from jax import lax
from jax.experimental import pallas as pl
from jax.experimental.pallas import tpu as pltpu
from jax.experimental.pallas import tpu_sc as plsc

NE = 8
TK = 2
BLK = 128
S = 2048
H = 768
I = 1536
NSLOT = S * TK
PAD = NSLOT + NE * BLK
NB = PAD // BLK
LANES = 128
WLANE = 128
NTILE = 32
RING = 2


def _router_body(x_ref, gw_ref, logits_ref, p1_ref, p2_ref, w1_ref, w2_ref,
                 be_ref, vb_ref):
    x = x_ref[...]
    gw = gw_ref[...]
    logits = lax.dot_general(x, gw, (((1,), (1,)), ((), ())),
                             preferred_element_type=jnp.float32)
    logits_ref[...] = logits
    m = jnp.max(logits, axis=-1, keepdims=True)
    ex = jnp.exp(logits - m)
    p = ex / jnp.sum(ex, axis=-1, keepdims=True)
    idx = lax.broadcasted_iota(jnp.int32, p.shape, 1)
    m1 = jnp.max(p, axis=-1, keepdims=True)
    i1 = jnp.min(jnp.where(p == m1, idx, NE), axis=-1, keepdims=True)
    pm = jnp.where(idx == i1, -1.0, p)
    m2 = jnp.max(pm, axis=-1, keepdims=True)
    i2 = jnp.min(jnp.where(pm == m2, idx, NE), axis=-1, keepdims=True)
    denom = m1 + m2
    w1 = m1 / denom
    w2 = m2 / denom

    oh1 = (idx == i1).astype(jnp.float32)
    oh2 = (idx == i2).astype(jnp.float32)
    a = oh1 + oh2
    r_io = lax.broadcasted_iota(jnp.int32, (BLK, BLK), 0)
    c_io = lax.broadcasted_iota(jnp.int32, (BLK, BLK), 1)
    tri = (r_io >= c_io).astype(jnp.float32)
    chunks = []
    run = jnp.zeros((1, NE), jnp.float32)
    for c in range(S // BLK):
        a_c = lax.slice(a, (c * BLK, 0), ((c + 1) * BLK, NE))
        cum_c = lax.dot_general(tri, a_c, (((1,), (0,)), ((), ())),
                                preferred_element_type=jnp.float32) + run
        chunks.append(cum_c)
        run = lax.slice(cum_c, (BLK - 1, 0), (BLK, NE))
    cum_incl = jnp.concatenate(chunks, axis=0)
    cum_excl = cum_incl - a
    counts = run
    padded = jnp.maximum(
        jnp.floor((counts + (BLK - 1)) * (1.0 / BLK)) * BLK, float(BLK))
    r8 = lax.broadcasted_iota(jnp.int32, (NE, NE), 0)
    c8 = lax.broadcasted_iota(jnp.int32, (NE, NE), 1)
    triu = (r8 < c8).astype(jnp.float32)
    pad_off = lax.dot_general(padded, triu, (((1,), (0,)), ((), ())),
                              preferred_element_type=jnp.float32)
    pos_base = pad_off + cum_excl
    pos1 = jnp.sum(oh1 * pos_base, axis=-1, keepdims=True)
    pos2 = jnp.sum(oh2 * pos_base, axis=-1, keepdims=True)

    p1_ref[...] = jnp.reshape(pos1.astype(jnp.int32), (NTILE, S // NTILE))
    p2_ref[...] = jnp.reshape(pos2.astype(jnp.int32), (NTILE, S // NTILE))
    w1_ref[...] = jnp.broadcast_to(w1, (S, WLANE))
    w2_ref[...] = jnp.broadcast_to(w2, (S, WLANE))
    pad_end = pad_off + padded
    starts = (lax.broadcasted_iota(jnp.int32, (64, 1), 0) * BLK
              ).astype(jnp.float32)
    be = jnp.sum((jnp.broadcast_to(pad_end, (64, NE)) <= starts)
                 .astype(jnp.float32), axis=1, keepdims=True)
    be_ref[...] = jnp.minimum(be, NE - 1).astype(jnp.int32)
    nv = jnp.sum(padded, axis=1, keepdims=True) * (1.0 / BLK)
    bix = (lax.broadcasted_iota(jnp.int32, (64, 1), 0)
           ).astype(jnp.float32)
    vb_ref[...] = jnp.minimum(bix, nv - 1.0).astype(jnp.int32)


def _run_router(x, gate_w):
    return pl.pallas_call(
        _router_body,
        out_shape=(
            jax.ShapeDtypeStruct((S, NE), jnp.float32),
            jax.ShapeDtypeStruct((NTILE, S // NTILE), jnp.int32),
            jax.ShapeDtypeStruct((NTILE, S // NTILE), jnp.int32),
            jax.ShapeDtypeStruct((S, WLANE), jnp.float32),
            jax.ShapeDtypeStruct((S, WLANE), jnp.float32),
            jax.ShapeDtypeStruct((64, 1), jnp.int32),
            jax.ShapeDtypeStruct((64, 1), jnp.int32),
        ),
    )(x, gate_w)


def _gmm_body(be_ref, vb_ref, xg_ref, gw_hbm, uw_hbm, dw_hbm, wcol_ref,
              out_ref, gbuf, ubuf, dbuf, gsem, usem, dsem):
    b = pl.program_id(0)
    e = be_ref[b, 0]
    prev_e = be_ref[jnp.maximum(b - 1, 0), 0]
    flo = jnp.where(b == 0, 0, prev_e + RING)
    fhi = jnp.minimum(e + RING, NE)
    wlo = jnp.where(b == 0, 0, prev_e + 1)
    whi = e + 1
    for k in range(NE):
        @pl.when((k >= flo) & (k < fhi))
        def _(k=k):
            pltpu.make_async_copy(gw_hbm.at[k], gbuf.at[k % RING],
                                  gsem.at[k % RING]).start()
            pltpu.make_async_copy(uw_hbm.at[k], ubuf.at[k % RING],
                                  usem.at[k % RING]).start()
            pltpu.make_async_copy(dw_hbm.at[k], dbuf.at[k % RING],
                                  dsem.at[k % RING]).start()
    for k in range(NE):
        @pl.when((k >= wlo) & (k < whi))
        def _(k=k):
            pltpu.make_async_copy(gw_hbm.at[k], gbuf.at[k % RING],
                                  gsem.at[k % RING]).wait()
            pltpu.make_async_copy(uw_hbm.at[k], ubuf.at[k % RING],
                                  usem.at[k % RING]).wait()
            pltpu.make_async_copy(dw_hbm.at[k], dbuf.at[k % RING],
                                  dsem.at[k % RING]).wait()

    @pl.when(vb_ref[b, 0] == b)
    def _():
        slot = e % RING
        x = xg_ref[...]
        gw = gbuf[slot]
        uw = ubuf[slot]
        dw = dbuf[slot]
        g = lax.dot_general(x, gw, (((1,), (1,)), ((), ())),
                            preferred_element_type=jnp.float32)
        u = lax.dot_general(x, uw, (((1,), (1,)), ((), ())),
                            preferred_element_type=jnp.float32)
        h = g * jax.nn.sigmoid(g) * u
        y = lax.dot_general(h, dw, (((1,), (1,)), ((), ())),
                            preferred_element_type=jnp.float32)
        out_ref[...] = y * wcol_ref[...][:, :1]


def _run_gmm(block_expert, vblock, xg, gate_proj_w, up_proj_w, down_proj_w,
             wmat):
    grid_spec = pltpu.PrefetchScalarGridSpec(
        num_scalar_prefetch=2,
        grid=(NB,),
        in_specs=[
            pl.BlockSpec((BLK, H), lambda b, be, vb: (vb[b, 0], 0)),
            pl.BlockSpec(memory_space=pl.ANY),
            pl.BlockSpec(memory_space=pl.ANY),
            pl.BlockSpec(memory_space=pl.ANY),
            pl.BlockSpec((BLK, WLANE), lambda b, be, vb: (vb[b, 0], 0)),
        ],
        out_specs=pl.BlockSpec((BLK, H), lambda b, be, vb: (vb[b, 0], 0)),
        scratch_shapes=[
            pltpu.VMEM((RING, I, H), jnp.float32),
            pltpu.VMEM((RING, I, H), jnp.float32),
            pltpu.VMEM((RING, H, I), jnp.float32),
            pltpu.SemaphoreType.DMA((RING,)),
            pltpu.SemaphoreType.DMA((RING,)),
            pltpu.SemaphoreType.DMA((RING,)),
        ],
    )
    return pl.pallas_call(
        _gmm_body,
        grid_spec=grid_spec,
        out_shape=jax.ShapeDtypeStruct((PAD, H), jnp.float32),
    )(block_expert, vblock, xg, gate_proj_w, up_proj_w, down_proj_w, wmat)


def _sc_dispatch(x, p1r, p2r, w1r, w2r):
    mesh = plsc.VectorSubcoreMesh(core_axis_name="c", subcore_axis_name="s")
    nw = mesh.num_cores * mesh.num_subcores
    t_per_w = S // nw

    @functools.partial(
        pl.kernel,
        out_type=(
            jax.ShapeDtypeStruct((PAD, H), jnp.float32),
            jax.ShapeDtypeStruct((PAD, WLANE), jnp.float32),
        ),
        mesh=mesh,
        name="sc_dispatch",
        scratch_types=[
            pltpu.VMEM((t_per_w, H), jnp.float32),
            pltpu.VMEM((t_per_w, WLANE), jnp.float32),
            pltpu.VMEM((t_per_w, WLANE), jnp.float32),
            pltpu.VMEM((t_per_w,), jnp.int32),
            pltpu.VMEM((t_per_w,), jnp.int32),
            pltpu.SemaphoreType.DMA,
            pltpu.SemaphoreType.DMA,
            pltpu.SemaphoreType.DMA,
            pltpu.SemaphoreType.DMA,
            pltpu.SemaphoreType.DMA,
        ],
    )
    def k(x_hbm, p1_hbm, p2_hbm, w1_hbm, w2_hbm, xg_hbm, wm_hbm,
          xrows_v, w1m_v, w2m_v, p1_v, p2_v,
          s1, s2, s3, s4, s5):
        wid = lax.axis_index("s") * mesh.num_cores + lax.axis_index("c")
        base = wid * t_per_w
        sl = pl.ds(base, t_per_w)
        i1 = pltpu.async_copy(x_hbm.at[sl], xrows_v, s1)
        i2 = pltpu.async_copy(p1_hbm.at[wid], p1_v, s2)
        i3 = pltpu.async_copy(p2_hbm.at[wid], p2_v, s3)
        i4 = pltpu.async_copy(w1_hbm.at[sl], w1m_v, s4)
        i5 = pltpu.async_copy(w2_hbm.at[sl], w2m_v, s5)
        i1.wait()
        i2.wait()
        i3.wait()
        i4.wait()
        i5.wait()
        c1 = pltpu.async_copy(xrows_v, xg_hbm.at[p1_v], s1)
        c2 = pltpu.async_copy(xrows_v, xg_hbm.at[p2_v], s2)
        c3 = pltpu.async_copy(w1m_v, wm_hbm.at[p1_v], s3)
        c4 = pltpu.async_copy(w2m_v, wm_hbm.at[p2_v], s4)
        c1.wait()
        c2.wait()
        c3.wait()
        c4.wait()

    return k(x, p1r, p2r, w1r, w2r)


def _sc_combine(osort, p1r, p2r):
    mesh = plsc.VectorSubcoreMesh(core_axis_name="c", subcore_axis_name="s")
    nw = mesh.num_cores * mesh.num_subcores
    t_per_w = S // nw
    csteps = H // 16

    @functools.partial(
        pl.kernel,
        out_type=jax.ShapeDtypeStruct((S, H), jnp.float32),
        mesh=mesh,
        name="sc_combine_rows",
        scratch_types=[
            pltpu.VMEM((t_per_w,), jnp.int32),
            pltpu.VMEM((t_per_w,), jnp.int32),
            pltpu.VMEM((t_per_w, H), jnp.float32),
            pltpu.VMEM((t_per_w, H), jnp.float32),
            pltpu.SemaphoreType.DMA,
        ],
    )
    def k(osort_hbm, p1_hbm, p2_hbm, out_hbm,
          i0_v, i1_v, a_v, b_v, sem):
        wid = lax.axis_index("s") * mesh.num_cores + lax.axis_index("c")
        base = wid * t_per_w
        sl = pl.ds(base, t_per_w)
        pltpu.sync_copy(p1_hbm.at[wid], i0_v)
        pltpu.sync_copy(p2_hbm.at[wid], i1_v)
        cp0 = pltpu.async_copy(osort_hbm.at[i0_v], a_v, sem)
        cp1 = pltpu.async_copy(osort_hbm.at[i1_v], b_v, sem)
        cp0.wait()
        cp1.wait()

        @plsc.parallel_loop(0, t_per_w * csteps, unroll=8)
        def _add(i):
            r = i // csteps
            c = i - r * csteps
            cs = pl.ds(c * 16, 16)
            a_v[r, cs] = a_v[r, cs] + b_v[r, cs]

        pltpu.sync_copy(a_v, out_hbm.at[sl])

    return k(osort, p1r, p2r)


def kernel(hidden_states, gate_w, gate_proj_w, up_proj_w, down_proj_w):
    batch, seq, hdim = hidden_states.shape
    x = hidden_states.reshape(-1, hdim)

    logits, p1r, p2r, w1r, w2r, block_expert, vblock = _run_router(x, gate_w)

    xg, wmat = _sc_dispatch(x, p1r, p2r, w1r, w2r)
    osort = _run_gmm(block_expert, vblock, xg, gate_proj_w, up_proj_w,
                     down_proj_w, wmat)
    final = _sc_combine(osort, p1r, p2r)
    return final.reshape(batch, seq, hdim), logits
